# Initial kernel scaffold; baseline (speedup 1.0000x reference)
import jax
import jax.numpy as jnp
from jax.experimental import pallas as pl


def kernel(seq, table):
    raise NotImplementedError("write your pallas kernel here")


# speedup vs baseline: 4.1279x; 4.1279x over previous
import jax
import jax.numpy as jnp
from jax.experimental import pallas as pl

DIM = 128
VOCAB = 6
B = 4096
L = 200
BB = 16


def _body(seq_ref, tab_ref, out_ref):
    s = seq_ref[...]
    t = tab_ref[...]
    acc = jnp.broadcast_to(t[:, :, 0:1], (BB, DIM, L))
    for v in range(1, VOCAB):
        acc = jnp.where(s == v, t[:, :, v : v + 1], acc)
    out_ref[...] = acc


def kernel(seq, table):
    seq3 = seq.astype(jnp.int32).reshape(B, 1, L)
    tabT = table.T.reshape(1, DIM, VOCAB)
    return pl.pallas_call(
        _body,
        grid=(B // BB,),
        in_specs=[
            pl.BlockSpec((BB, 1, L), lambda i: (i, 0, 0)),
            pl.BlockSpec((1, DIM, VOCAB), lambda i: (0, 0, 0)),
        ],
        out_specs=pl.BlockSpec((BB, DIM, L), lambda i: (i, 0, 0)),
        out_shape=jax.ShapeDtypeStruct((B, DIM, L), jnp.float32),
    )(seq3, tabT)
